# baseline (speedup 1.0000x reference)
import jax
import jax.numpy as jnp
from jax import lax
from jax.experimental import pallas as pl
from jax.experimental.pallas import tpu as pltpu, tpu_sc as plsc

R, C = 128, 32768
NC, NS = 2, 16
NW = NC * NS
ROWS_PER_W = R // NW
LANES = 16
NVEC = C // LANES
INT_MAX = 2147483647


def _argmax_body(x_hbm, out_hbm, buf0, buf1, res_buf, sem0, sem1):
    c = lax.axis_index("c")
    s = lax.axis_index("s")
    wid = s * NC + c
    row0 = wid * ROWS_PER_W
    bufs = (buf0, buf1)
    sems = (sem0, sem1)
    lane = lax.iota(jnp.int32, LANES)

    copies = [None, None]
    copies[0] = pltpu.make_async_copy(x_hbm.at[row0], buf0, sem0)
    copies[0].start()

    res = jnp.zeros((LANES,), jnp.int32)
    for k in range(ROWS_PER_W):
        if k + 1 < ROWS_PER_W:
            nxt = (k + 1) % 2
            copies[nxt] = pltpu.make_async_copy(
                x_hbm.at[row0 + k + 1], bufs[nxt], sems[nxt])
            copies[nxt].start()
        copies[k % 2].wait()
        buf = bufs[k % 2]

        def step(i, carry):
            best, biter = carry
            v = buf[pl.ds(i * LANES, LANES)]
            m = v > best
            best = jnp.where(m, v, best)
            biter = jnp.where(m, jnp.full((LANES,), i, jnp.int32), biter)
            return best, biter

        best0 = jnp.full((LANES,), -jnp.inf, jnp.float32)
        biter0 = jnp.zeros((LANES,), jnp.int32)
        best, biter = lax.fori_loop(0, NVEC, step, (best0, biter0),
                                    unroll=8)

        gmaxv = best
        for c in (1, 2, 4, 8):
            gmaxv = jnp.maximum(
                gmaxv, gmaxv.at[lane ^ c].get(mode="promise_in_bounds"))
        idx = biter * LANES + lane
        cand = jnp.where(best == gmaxv, idx,
                         jnp.full((LANES,), INT_MAX, jnp.int32))
        for c in (1, 2, 4, 8):
            cand = jnp.minimum(
                cand, cand.at[lane ^ c].get(mode="promise_in_bounds"))
        res = jnp.where(lane == k, cand, res)

    res_buf[...] = res
    pltpu.sync_copy(res_buf, out_hbm.at[wid])


def kernel(x):
    mesh = plsc.VectorSubcoreMesh(core_axis_name="c", subcore_axis_name="s")
    out = pl.kernel(
        _argmax_body,
        out_type=jax.ShapeDtypeStruct((NW, LANES), jnp.int32),
        mesh=mesh,
        scratch_types=[
            pltpu.VMEM((C,), jnp.float32),
            pltpu.VMEM((C,), jnp.float32),
            pltpu.VMEM((LANES,), jnp.int32),
            pltpu.SemaphoreType.DMA,
            pltpu.SemaphoreType.DMA,
        ],
    )(x)
    return out[:, :ROWS_PER_W].reshape(R)


# speedup vs baseline: 1.1046x; 1.1046x over previous
import jax
import jax.numpy as jnp
from jax import lax
from jax.experimental import pallas as pl
from jax.experimental.pallas import tpu as pltpu, tpu_sc as plsc

R, C = 128, 32768
NC, NS = 2, 16
NW = NC * NS
ROWS_PER_W = R // NW
LANES = 16
NVEC = C // LANES
NCHAIN = 4
SPAN = NVEC // NCHAIN
INT_MAX = 2147483647


def _argmax_body(x_hbm, out_hbm, buf0, buf1, res_buf, sem0, sem1):
    c = lax.axis_index("c")
    s = lax.axis_index("s")
    wid = s * NC + c
    row0 = wid * ROWS_PER_W
    bufs = (buf0, buf1)
    sems = (sem0, sem1)
    lane = lax.iota(jnp.int32, LANES)

    copies = [None, None]
    copies[0] = pltpu.make_async_copy(x_hbm.at[row0], buf0, sem0)
    copies[0].start()

    res = jnp.zeros((LANES,), jnp.int32)
    for k in range(ROWS_PER_W):
        if k + 1 < ROWS_PER_W:
            nxt = (k + 1) % 2
            copies[nxt] = pltpu.make_async_copy(
                x_hbm.at[row0 + k + 1], bufs[nxt], sems[nxt])
            copies[nxt].start()
        copies[k % 2].wait()
        buf = bufs[k % 2]

        def step(i, carry):
            bests, biters = carry
            nb, ni = [], []
            for j in range(NCHAIN):
                vi = i + j * SPAN
                v = buf[pl.ds(vi * LANES, LANES)]
                m = v > bests[j]
                nb.append(jnp.where(m, v, bests[j]))
                ni.append(jnp.where(m, jnp.full((LANES,), vi, jnp.int32),
                                    biters[j]))
            return tuple(nb), tuple(ni)

        bests0 = tuple(jnp.full((LANES,), -jnp.inf, jnp.float32)
                       for _ in range(NCHAIN))
        biters0 = tuple(jnp.zeros((LANES,), jnp.int32)
                        for _ in range(NCHAIN))
        bests, biters = lax.fori_loop(0, SPAN, step, (bests0, biters0),
                                      unroll=4)

        best, biter = bests[0], biters[0]
        for j in range(1, NCHAIN):
            m = bests[j] > best
            best = jnp.where(m, bests[j], best)
            biter = jnp.where(m, biters[j], biter)

        gmaxv = best
        for c in (1, 2, 4, 8):
            gmaxv = jnp.maximum(
                gmaxv, gmaxv.at[lane ^ c].get(mode="promise_in_bounds"))
        idx = biter * LANES + lane
        cand = jnp.where(best == gmaxv, idx,
                         jnp.full((LANES,), INT_MAX, jnp.int32))
        for c in (1, 2, 4, 8):
            cand = jnp.minimum(
                cand, cand.at[lane ^ c].get(mode="promise_in_bounds"))
        res = jnp.where(lane == k, cand, res)

    res_buf[...] = res
    pltpu.sync_copy(res_buf, out_hbm.at[wid])


def kernel(x):
    mesh = plsc.VectorSubcoreMesh(core_axis_name="c", subcore_axis_name="s")
    out = pl.kernel(
        _argmax_body,
        out_type=jax.ShapeDtypeStruct((NW, LANES), jnp.int32),
        mesh=mesh,
        scratch_types=[
            pltpu.VMEM((C,), jnp.float32),
            pltpu.VMEM((C,), jnp.float32),
            pltpu.VMEM((LANES,), jnp.int32),
            pltpu.SemaphoreType.DMA,
            pltpu.SemaphoreType.DMA,
        ],
    )(x)
    return out[:, :ROWS_PER_W].reshape(R)


# speedup vs baseline: 1.1820x; 1.0700x over previous
import jax
import jax.numpy as jnp
from jax import lax
from jax.experimental import pallas as pl
from jax.experimental.pallas import tpu as pltpu, tpu_sc as plsc

R, C = 128, 32768
NC, NS = 2, 16
NW = NC * NS
ROWS_PER_W = R // NW
LANES = 16
NVEC = C // LANES
NCHUNK = 2
CELEM = C // NCHUNK
CVEC = NVEC // NCHUNK
NCHAIN = 4
SPANC = CVEC // NCHAIN
NBUF = 3
NQ = ROWS_PER_W * NCHUNK
INT_MAX = 2147483647


def _argmax_body(x_hbm, out_hbm, b0, b1, b2, res_buf, stage_v, out64,
                 shared, sem0, sem1, sem2):
    c = lax.axis_index("c")
    s = lax.axis_index("s")
    wid = c * NS + s
    row0 = wid * ROWS_PER_W
    lane = lax.iota(jnp.int32, LANES)
    bufs = (b0, b1, b2)
    sems = (sem0, sem1, sem2)
    copies = [None] * NBUF

    def start(q):
        k, h = divmod(q, NCHUNK)
        cp = pltpu.make_async_copy(
            x_hbm.at[row0 + k, pl.ds(h * CELEM, CELEM)],
            bufs[q % NBUF], sems[q % NBUF])
        cp.start()
        copies[q % NBUF] = cp

    for q in range(NBUF):
        start(q)

    res = jnp.zeros((LANES,), jnp.int32)
    rbest = None
    rbiter = None
    for q in range(NQ):
        copies[q % NBUF].wait()
        buf = bufs[q % NBUF]
        k, h = divmod(q, NCHUNK)

        def step(i, carry, _buf=buf, _h=h):
            bests, biters = carry
            nb, ni = [], []
            for j in range(NCHAIN):
                vi = i + j * SPANC
                v = _buf[pl.ds(vi * LANES, LANES)]
                m = v > bests[j]
                nb.append(jnp.where(m, v, bests[j]))
                gvi = vi + _h * CVEC
                ni.append(jnp.where(m, jnp.full((LANES,), gvi, jnp.int32),
                                    biters[j]))
            return tuple(nb), tuple(ni)

        bests0 = tuple(jnp.full((LANES,), -jnp.inf, jnp.float32)
                       for _ in range(NCHAIN))
        biters0 = tuple(jnp.zeros((LANES,), jnp.int32)
                        for _ in range(NCHAIN))
        bests, biters = lax.fori_loop(0, SPANC, step, (bests0, biters0),
                                      unroll=4)

        if h == 0:
            rbest, rbiter = bests[0], biters[0]
            rest = range(1, NCHAIN)
        else:
            rest = range(NCHAIN)
        for j in rest:
            m = bests[j] > rbest
            rbest = jnp.where(m, bests[j], rbest)
            rbiter = jnp.where(m, biters[j], rbiter)

        if h == NCHUNK - 1:
            gmaxv = rbest
            for step2 in (1, 2, 4, 8):
                gmaxv = jnp.maximum(
                    gmaxv, gmaxv.at[lane ^ step2].get(mode="promise_in_bounds"))
            idx = rbiter * LANES + lane
            cand = jnp.where(rbest == gmaxv, idx,
                             jnp.full((LANES,), INT_MAX, jnp.int32))
            for step2 in (1, 2, 4, 8):
                cand = jnp.minimum(
                    cand, cand.at[lane ^ step2].get(mode="promise_in_bounds"))
            res = jnp.where(lane == k, cand, res)

        if q + NBUF < NQ:
            start(q + NBUF)

    res_buf[...] = res
    pltpu.sync_copy(res_buf, shared.at[pl.ds(s * LANES, LANES)])
    plsc.subcore_barrier()

    @pl.when(s == 0)
    def _():
        pltpu.sync_copy(shared, stage_v)
        lm = jnp.bitwise_and(lane, 3)
        for t in range(NS // ROWS_PER_W):
            g = []
            for i in range(ROWS_PER_W):
                w = stage_v[pl.ds((ROWS_PER_W * t + i) * LANES, LANES)]
                g.append(w.at[lm].get(mode="promise_in_bounds"))
            sel = jnp.where(lane < 4, g[0],
                            jnp.where(lane < 8, g[1],
                                      jnp.where(lane < 12, g[2], g[3])))
            out64[pl.ds(t * LANES, LANES)] = sel
        pltpu.sync_copy(out64, out_hbm.at[pl.ds(c * (NS * ROWS_PER_W),
                                                NS * ROWS_PER_W)])


def kernel(x):
    mesh = plsc.VectorSubcoreMesh(core_axis_name="c", subcore_axis_name="s")
    return pl.kernel(
        _argmax_body,
        out_type=jax.ShapeDtypeStruct((R,), jnp.int32),
        mesh=mesh,
        scratch_types=[
            pltpu.VMEM((CELEM,), jnp.float32),
            pltpu.VMEM((CELEM,), jnp.float32),
            pltpu.VMEM((CELEM,), jnp.float32),
            pltpu.VMEM((LANES,), jnp.int32),
            pltpu.VMEM((NS * LANES,), jnp.int32),
            pltpu.VMEM((NS * ROWS_PER_W,), jnp.int32),
            pltpu.VMEM_SHARED((NS * LANES,), jnp.int32),
            pltpu.SemaphoreType.DMA,
            pltpu.SemaphoreType.DMA,
            pltpu.SemaphoreType.DMA,
        ],
    )(x)
